# Initial kernel scaffold; baseline (speedup 1.0000x reference)
import jax
import jax.numpy as jnp
from jax.experimental import pallas as pl


def kernel(x, y, x_normals):
    raise NotImplementedError("write your pallas kernel here")


# speedup vs baseline: 6.4038x; 6.4038x over previous
import functools

import jax
import jax.numpy as jnp
from jax.experimental import pallas as pl

_P1, _P2 = 778, 3000
_P1P, _P2P = 784, 3072
_PAD = 1e17


def _nn_body(x_ref, y_ref, n_ref, y2x_ref, x2y_ref, yidx_ref):
    xb = x_ref[0]
    yb = y_ref[0]
    nb = n_ref[0]

    d = None
    g = None
    for c in range(3):
        xc = xb[:, c : c + 1]
        yc = yb[c : c + 1, :]
        diff = yc - xc
        sq = diff * diff
        gc = nb[:, c : c + 1] * diff
        d = sq if d is None else d + sq
        g = gc if g is None else g + gc

    row_min = jnp.min(d, axis=1, keepdims=True)
    x2y_ref[0] = jnp.sqrt(row_min)

    col_min = jnp.min(d, axis=0, keepdims=True)
    iota = jax.lax.broadcasted_iota(jnp.int32, d.shape, 0)
    yidx = jnp.min(jnp.where(d == col_min, iota, _P1P), axis=0, keepdims=True)
    yidx_ref[0] = yidx
    dotv = jnp.sum(jnp.where(iota == yidx, g, 0.0), axis=0, keepdims=True)
    y2x_ref[0] = jnp.sqrt(col_min) * jnp.sign(dotv)


@functools.partial(jax.jit, static_argnames=())
def kernel(x, y, x_normals):
    B = x.shape[0]
    xp = jnp.pad(x, ((0, 0), (0, _P1P - _P1), (0, 0)), constant_values=_PAD)
    npad = jnp.pad(x_normals, ((0, 0), (0, _P1P - _P1), (0, 0)))
    yt = jnp.pad(
        jnp.transpose(y, (0, 2, 1)), ((0, 0), (0, 0), (0, _P2P - _P2)),
        constant_values=_PAD,
    )

    y2x_s, x2y_s, yidx = pl.pallas_call(
        _nn_body,
        grid=(B,),
        in_specs=[
            pl.BlockSpec((1, _P1P, 3), lambda b: (b, 0, 0)),
            pl.BlockSpec((1, 3, _P2P), lambda b: (b, 0, 0)),
            pl.BlockSpec((1, _P1P, 3), lambda b: (b, 0, 0)),
        ],
        out_specs=[
            pl.BlockSpec((1, 1, _P2P), lambda b: (b, 0, 0)),
            pl.BlockSpec((1, _P1P, 1), lambda b: (b, 0, 0)),
            pl.BlockSpec((1, 1, _P2P), lambda b: (b, 0, 0)),
        ],
        out_shape=[
            jax.ShapeDtypeStruct((B, 1, _P2P), jnp.float32),
            jax.ShapeDtypeStruct((B, _P1P, 1), jnp.float32),
            jax.ShapeDtypeStruct((B, 1, _P2P), jnp.int32),
        ],
    )(xp, yt, npad)

    return (
        y2x_s[:, 0, :_P2],
        x2y_s[:, :_P1, 0],
        yidx[:, 0, :_P2],
    )
